# Initial kernel scaffold; baseline (speedup 1.0000x reference)
import jax
import jax.numpy as jnp
from jax.experimental import pallas as pl


def kernel(x, weight, gate_w, gate_b):
    raise NotImplementedError("write your pallas kernel here")


# speedup vs baseline: 1.8706x; 1.8706x over previous
import functools

import jax
import jax.numpy as jnp
from jax.experimental import pallas as pl
from jax.experimental.pallas import tpu as pltpu


def _moe_body(x_ref, w_ref, gw_ref, gb_ref, out_ref, g_scr, *, bm, bn, n_exp):
    n = pl.program_id(1)
    e = pl.program_id(2)

    @pl.when(jnp.logical_and(n == 0, e == 0))
    def _gating():
        xb = x_ref[...]
        logits = jnp.dot(xb, gw_ref[...], preferred_element_type=jnp.float32)
        logits = logits + gb_ref[...]
        mx = jnp.max(logits, axis=-1, keepdims=True)
        p = jnp.exp(logits - mx)
        p = p / jnp.sum(p, axis=-1, keepdims=True)
        ii = jax.lax.broadcasted_iota(jnp.int32, (bm, n_exp), 1)
        pmax = jnp.max(p, axis=-1, keepdims=True)
        cand = jnp.where(p == pmax, ii, n_exp)
        amin = jnp.min(cand, axis=-1, keepdims=True)
        onehot = (ii == amin).astype(jnp.float32)
        blk = jnp.max(onehot, axis=0, keepdims=True)
        g_scr[...] = p * blk

    y = jnp.dot(x_ref[...], w_ref[...], preferred_element_type=jnp.float32)
    g_all = g_scr[...]
    lane = jax.lax.broadcasted_iota(jnp.int32, (bm, n_exp), 1)
    g = jnp.sum(jnp.where(lane == e, g_all, 0.0), axis=1, keepdims=True)
    contrib = y * g

    @pl.when(e == 0)
    def _init():
        out_ref[...] = contrib

    @pl.when(e > 0)
    def _acc():
        out_ref[...] += contrib


def kernel(x, weight, gate_w, gate_b):
    B, d_model = x.shape
    n_exp = gate_w.shape[1]
    bm = d_model
    assert B % bm == 0
    n_row_blocks = B // bm
    bn = min(512, d_model)
    n_tiles = d_model // bn

    gb2 = gate_b.reshape(1, n_exp)

    body = functools.partial(_moe_body, bm=bm, bn=bn, n_exp=n_exp)
    out = pl.pallas_call(
        body,
        grid=(n_row_blocks, n_tiles, n_exp),
        in_specs=[
            pl.BlockSpec((bm, d_model), lambda m, n, e: (m, 0)),
            pl.BlockSpec((d_model, bn), lambda m, n, e: (0, e * (d_model // bn) + n)),
            pl.BlockSpec((d_model, n_exp), lambda m, n, e: (0, 0)),
            pl.BlockSpec((1, n_exp), lambda m, n, e: (0, 0)),
        ],
        out_specs=pl.BlockSpec((bm, bn), lambda m, n, e: (m, n)),
        out_shape=jax.ShapeDtypeStruct((B, d_model), jnp.float32),
        scratch_shapes=[pltpu.VMEM((bm, n_exp), jnp.float32)],
    )(x, weight, gate_w, gb2)
    return out
